# baseline (speedup 1.0000x reference)
import jax
import jax.numpy as jnp
from jax import lax
from jax.experimental import pallas as pl
from jax.experimental.pallas import tpu as pltpu
from jax.experimental.pallas import tpu_sc as plsc

_B, _NH, _T = 4, 16, 8192
_S = 256
_K = 5
_NSUB = 16
_NCORE = 2
_CPT = _T // _NSUB
_BPC = _B // _NCORE
_NQ = _CPT // 128
_NR = _CPT // 128

_mesh = plsc.VectorSubcoreMesh(
    core_axis_name="c", subcore_axis_name="s",
    num_cores=_NCORE, num_subcores=_NSUB,
)


def _sc_body(attn, map2d, part, att_l, map_l, tv0, tv1, zb, acc0, acc1,
             d0, d1, d2, d3, dm, dsc):
    cid = lax.axis_index("c")
    sid = lax.axis_index("s")
    base = sid * _CPT
    rows0 = cid * (_BPC * _NH)

    dsems = (d0, d1, d2, d3)
    in_copies = []
    for q in range(_NQ):
        in_copies.append(pltpu.async_copy(
            attn.at[pl.ds(rows0, _BPC * _NH), pl.ds(base + q * 128, 128)],
            att_l.at[:, pl.ds(q * 128, 128)], dsems[q]))
    map_copy = pltpu.async_copy(
        map2d.at[pl.ds(sid * _NR, _NR)], map_l, dm)

    @pl.when(sid == 0)
    def _init():
        for i in range(_S // 16):
            zb[pl.ds(i * 16, 16)] = jnp.zeros((16,), jnp.float32)
        pltpu.sync_copy(zb, acc0)
        pltpu.sync_copy(zb, acc1)

    plsc.subcore_barrier()
    map_copy.wait()

    sc_copies = []
    for q in range(_NQ):
        in_copies[q].wait()

        @plsc.parallel_loop(q * 128, (q + 1) * 128, step=16, unroll=2)
        def g_body(t0, q=q):
            for b, tv in ((0, tv0), (1, tv1)):
                vs = [att_l[b * _NH + h, pl.ds(t0, 16)]
                      for h in range(_NH)]
                while len(vs) > 1:
                    vs = [vs[i] + vs[i + 1] for i in range(0, len(vs), 2)]
                tv[pl.ds(t0, 16)] = vs[0] * jnp.float32(1.0 / _NH)
        for tv, acc in ((tv0, acc0), (tv1, acc1)):
            sc_copies.append(pltpu.async_copy(
                tv.at[pl.ds(q * 128, 128)], acc.at[map_l.at[q]], dsc,
                add=True))
    for c in sc_copies:
        c.wait()
    plsc.subcore_barrier()

    @pl.when(sid == 0)
    def _out():
        pltpu.sync_copy(acc0, part.at[cid * _BPC])
        pltpu.sync_copy(acc1, part.at[cid * _BPC + 1])


def _tc_body(part_ref, vals_ref, idx_ref):
    work = part_ref[...]
    total = jnp.sum(work, axis=-1, keepdims=True)
    col = lax.broadcasted_iota(jnp.int32, (_B, 8), 1)
    sent = lax.broadcasted_iota(jnp.int32, (_B, _S), 1)
    vals_acc = jnp.zeros((_B, 8), jnp.float32)
    idx_acc = jnp.zeros((_B, 8), jnp.int32)
    for i in range(_K):
        mx = jnp.max(work, axis=-1, keepdims=True)
        cand = jnp.where(work == mx, sent, jnp.int32(1 << 30))
        ind = jnp.min(cand, axis=-1, keepdims=True)
        vals_acc = jnp.where(col == i, mx, vals_acc)
        idx_acc = jnp.where(col == i, ind, idx_acc)
        work = jnp.where(sent == ind, jnp.float32(-1.0), work)
    vals_ref[...] = (vals_acc / total)[:, :_K]
    idx_ref[...] = idx_acc[:, :_K]


def kernel(attention_weights, token_to_sentence_map):
    attn2d = attention_weights.reshape(_B * _NH, _T)
    map2d = token_to_sentence_map.astype(jnp.int32).reshape(_T // 128, 128)
    sc_run = pl.kernel(
        _sc_body,
        out_type=jax.ShapeDtypeStruct((_B, _S), jnp.float32),
        mesh=_mesh,
        compiler_params=pltpu.CompilerParams(needs_layout_passes=False),
        scratch_types=[
            pltpu.VMEM((_BPC * _NH, _CPT), jnp.float32),
            pltpu.VMEM((_NR, 128), jnp.int32),
            pltpu.VMEM((_CPT,), jnp.float32),
            pltpu.VMEM((_CPT,), jnp.float32),
            pltpu.VMEM((_S,), jnp.float32),
            pltpu.VMEM_SHARED((_S,), jnp.float32),
            pltpu.VMEM_SHARED((_S,), jnp.float32),
        ] + [pltpu.SemaphoreType.DMA] * 6,
    )
    part = sc_run(attn2d, map2d)
    vals, idx = pl.pallas_call(
        _tc_body,
        out_shape=[
            jax.ShapeDtypeStruct((_B, _K), jnp.float32),
            jax.ShapeDtypeStruct((_B, _K), jnp.int32),
        ],
    )(part)
    return vals, idx


# speedup vs baseline: 1.0535x; 1.0535x over previous
import jax
import jax.numpy as jnp
from jax import lax
from jax.experimental import pallas as pl
from jax.experimental.pallas import tpu as pltpu
from jax.experimental.pallas import tpu_sc as plsc

_B, _NH, _T = 4, 16, 8192
_S = 256
_K = 5
_NSUB = 16
_NCORE = 2
_CPT = _T // _NSUB
_BPC = _B // _NCORE
_NQ = _CPT // 128
_NR = _CPT // 128

_mesh = plsc.VectorSubcoreMesh(
    core_axis_name="c", subcore_axis_name="s",
    num_cores=_NCORE, num_subcores=_NSUB,
)


def _sc_body(attn, map2d, part, att_l, map_l, tv0, tv1, zb, acc0, acc1,
             d0, d1, d2, d3, dm, dsc):
    cid = lax.axis_index("c")
    sid = lax.axis_index("s")
    base = sid * _CPT
    rows0 = cid * (_BPC * _NH)

    dsems = (d0, d1, d2, d3)
    in_copies = []
    for q in range(_NQ):
        in_copies.append(pltpu.async_copy(
            attn.at[pl.ds(rows0, _BPC * _NH), pl.ds(base + q * 128, 128)],
            att_l.at[:, pl.ds(q * 128, 128)], dsems[q]))
    map_copy = pltpu.async_copy(
        map2d.at[pl.ds(sid * _NR, _NR)], map_l, dm)

    @pl.when(sid == 0)
    def _init():
        for i in range(_S // 16):
            zb[pl.ds(i * 16, 16)] = jnp.zeros((16,), jnp.float32)
        pltpu.sync_copy(zb, acc0)
        pltpu.sync_copy(zb, acc1)

    plsc.subcore_barrier()
    map_copy.wait()

    sc_copies = []
    for q in range(_NQ):
        in_copies[q].wait()

        def g_body(g, carry, q=q):
            t0 = q * 128 + g * 16
            for b, tv in ((0, tv0), (1, tv1)):
                vs = [att_l[b * _NH + h, pl.ds(t0, 16)]
                      for h in range(_NH)]
                while len(vs) > 1:
                    vs = [vs[i] + vs[i + 1] for i in range(0, len(vs), 2)]
                tv[pl.ds(t0, 16)] = vs[0] * jnp.float32(1.0 / _NH)
            return carry

        lax.fori_loop(0, 8, g_body, 0)
        for tv, acc in ((tv0, acc0), (tv1, acc1)):
            sc_copies.append(pltpu.async_copy(
                tv.at[pl.ds(q * 128, 128)], acc.at[map_l.at[q]], dsc,
                add=True))
    for c in sc_copies:
        c.wait()
    plsc.subcore_barrier()

    @pl.when(sid == 0)
    def _out():
        pltpu.sync_copy(acc0, part.at[cid * _BPC])
        pltpu.sync_copy(acc1, part.at[cid * _BPC + 1])


def _tc_body(part_ref, vals_ref, idx_ref):
    work = part_ref[...]
    total = jnp.sum(work, axis=-1, keepdims=True)
    col = lax.broadcasted_iota(jnp.int32, (_B, 8), 1)
    sent = lax.broadcasted_iota(jnp.int32, (_B, _S), 1)
    vals_acc = jnp.zeros((_B, 8), jnp.float32)
    idx_acc = jnp.zeros((_B, 8), jnp.int32)
    for i in range(_K):
        mx = jnp.max(work, axis=-1, keepdims=True)
        cand = jnp.where(work == mx, sent, jnp.int32(1 << 30))
        ind = jnp.min(cand, axis=-1, keepdims=True)
        vals_acc = jnp.where(col == i, mx, vals_acc)
        idx_acc = jnp.where(col == i, ind, idx_acc)
        work = jnp.where(sent == ind, jnp.float32(-1.0), work)
    vals_ref[...] = (vals_acc / total)[:, :_K]
    idx_ref[...] = idx_acc[:, :_K]


def kernel(attention_weights, token_to_sentence_map):
    attn2d = attention_weights.reshape(_B * _NH, _T)
    map2d = token_to_sentence_map.astype(jnp.int32).reshape(_T // 128, 128)
    sc_run = pl.kernel(
        _sc_body,
        out_type=jax.ShapeDtypeStruct((_B, _S), jnp.float32),
        mesh=_mesh,
        compiler_params=pltpu.CompilerParams(needs_layout_passes=False),
        scratch_types=[
            pltpu.VMEM((_BPC * _NH, _CPT), jnp.float32),
            pltpu.VMEM((_NR, 128), jnp.int32),
            pltpu.VMEM((_CPT,), jnp.float32),
            pltpu.VMEM((_CPT,), jnp.float32),
            pltpu.VMEM((_S,), jnp.float32),
            pltpu.VMEM_SHARED((_S,), jnp.float32),
            pltpu.VMEM_SHARED((_S,), jnp.float32),
        ] + [pltpu.SemaphoreType.DMA] * 6,
    )
    part = sc_run(attn2d, map2d)
    vals, idx = pl.pallas_call(
        _tc_body,
        out_shape=[
            jax.ShapeDtypeStruct((_B, _K), jnp.float32),
            jax.ShapeDtypeStruct((_B, _K), jnp.int32),
        ],
    )(part)
    return vals, idx
